# baseline (speedup 1.0000x reference)
import functools

import jax
import jax.numpy as jnp
from jax import lax
from jax.experimental import pallas as pl
from jax.experimental.pallas import tpu as pltpu
from jax.experimental.pallas import tpu_sc as plsc

O, I, K = 4096, 512, 2
CBS, G = 256, 8
OUT_W = I * G
NW = 32
ROWS_PER_W = O // NW
CHUNKS = OUT_W // 16


BLK = 4
NBLK = ROWS_PER_W // BLK


def _body(cb_hbm, codes_hbm, out_hbm, tab_v,
          codes_v0, codes_v1, out_v0, out_v1,
          sem_in0, sem_in1, sem_out0, sem_out1):
    wid = lax.axis_index("s") * 2 + lax.axis_index("c")
    row0 = wid * ROWS_PER_W
    pltpu.sync_copy(cb_hbm, tab_v)

    codes_bufs = (codes_v0, codes_v1)
    out_bufs = (out_v0, out_v1)
    sems_in = (sem_in0, sem_in1)
    sems_out = (sem_out0, sem_out1)

    lanes = lax.iota(jnp.int32, 16)
    col = lanes & 7
    pat = (lanes >> 3) * 2
    zeros = lanes * 0

    def start_in(b):
        return pltpu.async_copy(
            codes_hbm.at[pl.ds(row0 + b * BLK, BLK)],
            codes_bufs[b % 2], sems_in[b % 2])

    def start_out(b):
        return pltpu.async_copy(
            out_bufs[b % 2],
            out_hbm.at[pl.ds(row0 + b * BLK, BLK)], sems_out[b % 2])

    def compute_block(b):
        codes_buf = codes_bufs[b % 2]
        out_buf = out_bufs[b % 2]

        def do_row(n, carry):
            @plsc.parallel_loop(0, CHUNKS // 4, unroll=4)
            def quad(q):
                cvec = codes_buf[n, pl.ds(q * 16, 16)]
                for t in range(4):
                    c0 = jnp.take_along_axis(
                        cvec, pat + 4 * t, axis=0, mode="promise_in_bounds")
                    c1 = jnp.take_along_axis(
                        cvec, pat + (4 * t + 1), axis=0,
                        mode="promise_in_bounds")
                    v0 = plsc.load_gather(tab_v, [c0, col])
                    v1 = plsc.load_gather(tab_v, [c1 + CBS, col])
                    out_buf[n, pl.ds((q * 4 + t) * 16, 16)] = v0 + v1

            return carry

        lax.fori_loop(0, BLK, do_row, 0)

    in_h = [None, None]
    out_h = [None, None]
    in_h[0] = start_in(0)
    for b in range(NBLK):
        cur = b % 2
        in_h[cur].wait()
        if b + 1 < NBLK:
            in_h[(b + 1) % 2] = start_in(b + 1)
        if out_h[cur] is not None:
            out_h[cur].wait()
        compute_block(b)
        out_h[cur] = start_out(b)
    out_h[0].wait()
    out_h[1].wait()


def kernel(codes, codebooks):
    flat_cb = codebooks.reshape(K * CBS, G)
    codes2d = codes.reshape(O, I * K)
    mesh = plsc.VectorSubcoreMesh(core_axis_name="c", subcore_axis_name="s")
    k = functools.partial(
        pl.kernel,
        mesh=mesh,
        out_type=jax.ShapeDtypeStruct((O, OUT_W), jnp.float32),
        scratch_types=[
            pltpu.VMEM((K * CBS, G), jnp.float32),
            pltpu.VMEM((BLK, I * K), jnp.int32),
            pltpu.VMEM((BLK, I * K), jnp.int32),
            pltpu.VMEM((BLK, OUT_W), jnp.float32),
            pltpu.VMEM((BLK, OUT_W), jnp.float32),
            pltpu.SemaphoreType.DMA,
            pltpu.SemaphoreType.DMA,
            pltpu.SemaphoreType.DMA,
            pltpu.SemaphoreType.DMA,
        ],
        compiler_params=pltpu.CompilerParams(needs_layout_passes=False),
    )(_body)
    return k(flat_cb, codes2d)


# speedup vs baseline: 1.2540x; 1.2540x over previous
import functools

import jax
import jax.numpy as jnp
from jax import lax
from jax.experimental import pallas as pl
from jax.experimental.pallas import tpu as pltpu
from jax.experimental.pallas import tpu_sc as plsc

O, I, K = 4096, 512, 2
CBS, G = 256, 8
OUT_W = I * G
NW = 32
ROWS_PER_W = O // NW
CHUNKS = OUT_W // 16


BLK = 4
NBLK = ROWS_PER_W // BLK


def _body(cb_hbm, codes_hbm, out_hbm, tab_v,
          codes_v0, codes_v1, out_v0, out_v1,
          sem_in0, sem_in1, sem_out0, sem_out1):
    wid = lax.axis_index("s") * 2 + lax.axis_index("c")
    row0 = wid * ROWS_PER_W
    pltpu.sync_copy(cb_hbm, tab_v)

    codes_bufs = (codes_v0, codes_v1)
    out_bufs = (out_v0, out_v1)
    sems_in = (sem_in0, sem_in1)
    sems_out = (sem_out0, sem_out1)

    lanes = lax.iota(jnp.int32, 16)
    col = lanes
    pat = (lanes >> 3) * 2
    zeros = lanes * 0

    def start_in(b):
        return pltpu.async_copy(
            codes_hbm.at[pl.ds(row0 + b * BLK, BLK)],
            codes_bufs[b % 2], sems_in[b % 2])

    def start_out(b):
        return pltpu.async_copy(
            out_bufs[b % 2],
            out_hbm.at[pl.ds(row0 + b * BLK, BLK)], sems_out[b % 2])

    def compute_block(b):
        codes_buf = codes_bufs[b % 2]
        out_buf = out_bufs[b % 2]

        def do_row(n, carry):
            @plsc.parallel_loop(0, CHUNKS // 4, unroll=4)
            def quad(q):
                cvec = codes_buf[n, pl.ds(q * 16, 16)]
                for t in range(4):
                    c0 = jnp.take_along_axis(
                        cvec, pat + 4 * t, axis=0, mode="promise_in_bounds")
                    c1 = jnp.take_along_axis(
                        cvec, pat + (4 * t + 1), axis=0,
                        mode="promise_in_bounds")
                    v0 = plsc.load_gather(tab_v, [c0, col])
                    v1 = plsc.load_gather(tab_v, [c1 + CBS, col])
                    out_buf[n, pl.ds((q * 4 + t) * 16, 16)] = v0 + v1

            return carry

        lax.fori_loop(0, BLK, do_row, 0)

    in_h = [None, None]
    out_h = [None, None]
    in_h[0] = start_in(0)
    for b in range(NBLK):
        cur = b % 2
        in_h[cur].wait()
        if b + 1 < NBLK:
            in_h[(b + 1) % 2] = start_in(b + 1)
        if out_h[cur] is not None:
            out_h[cur].wait()
        compute_block(b)
        out_h[cur] = start_out(b)
    out_h[0].wait()
    out_h[1].wait()


def kernel(codes, codebooks):
    flat_cb = codebooks.reshape(K * CBS, G)
    flat_cb = jnp.concatenate([flat_cb, flat_cb], axis=1)
    codes2d = codes.reshape(O, I * K)
    mesh = plsc.VectorSubcoreMesh(core_axis_name="c", subcore_axis_name="s")
    k = functools.partial(
        pl.kernel,
        mesh=mesh,
        out_type=jax.ShapeDtypeStruct((O, OUT_W), jnp.float32),
        scratch_types=[
            pltpu.VMEM((K * CBS, 2 * G), jnp.float32),
            pltpu.VMEM((BLK, I * K), jnp.int32),
            pltpu.VMEM((BLK, I * K), jnp.int32),
            pltpu.VMEM((BLK, OUT_W), jnp.float32),
            pltpu.VMEM((BLK, OUT_W), jnp.float32),
            pltpu.SemaphoreType.DMA,
            pltpu.SemaphoreType.DMA,
            pltpu.SemaphoreType.DMA,
            pltpu.SemaphoreType.DMA,
        ],
        compiler_params=pltpu.CompilerParams(needs_layout_passes=False),
    )(_body)
    return k(flat_cb, codes2d)
